# baseline (speedup 1.0000x reference)
import jax
import jax.numpy as jnp
from jax.experimental import pallas as pl

N = 5000
PRED = 85
NCLS = 80
NPAD = 5120
TI = 256
TA = 640
IOU_THR = 0.5


def _decode_kernel(p_ref, out_ref):
    p = p_ref[...]
    cx = p[:, 0:1]
    cy = p[:, 1:2]
    w = p[:, 2:3]
    h = p[:, 3:4]
    conf = p[:, 4:5]
    cls = p[:, 5:PRED]
    m = jnp.max(cls, axis=1, keepdims=True)
    iota = jax.lax.broadcasted_iota(jnp.int32, cls.shape, 1)
    amax = jnp.min(jnp.where(cls == m, iota, NCLS), axis=1, keepdims=True)
    x1 = cx - w * 0.5
    y1 = cy - h * 0.5
    x2 = cx + w * 0.5
    y2 = cy + h * 0.5
    score = conf * m
    area = jnp.maximum(x2 - x1, 0.0) * jnp.maximum(y2 - y1, 0.0)
    zero = jnp.zeros_like(score)
    out_ref[...] = jnp.concatenate(
        [x1, y1, x2, y2, score, amax.astype(jnp.float32), area, zero], axis=1
    )


def _nms_kernel(rows_ref, cols_ref, out_ref):
    r = rows_ref[...]
    c = cols_ref[...]
    xi1 = r[:, 0:1]
    yi1 = r[:, 1:2]
    xi2 = r[:, 2:3]
    yi2 = r[:, 3:4]
    si = r[:, 4:5]
    ai = r[:, 6:7]
    xj1 = c[0:1, :]
    yj1 = c[1:2, :]
    xj2 = c[2:3, :]
    yj2 = c[3:4, :]
    sj = c[4:5, :]
    aj = c[6:7, :]
    ix1 = jnp.maximum(xi1, xj1)
    iy1 = jnp.maximum(yi1, yj1)
    ix2 = jnp.minimum(xi2, xj2)
    iy2 = jnp.minimum(yi2, yj2)
    iw = jnp.maximum(ix2 - ix1, 0.0)
    ih = jnp.maximum(iy2 - iy1, 0.0)
    inter = iw * ih
    union = (ai + aj) - inter
    iou = inter / jnp.maximum(union, 1e-9)
    ii = pl.program_id(0) * TI + jax.lax.broadcasted_iota(jnp.int32, (TI, 1), 0)
    jj = jax.lax.broadcasted_iota(jnp.int32, (1, NPAD), 1)
    higher = (sj > si) | ((sj == si) & (jj < ii))
    supp = jnp.any(higher & (iou > IOU_THR), axis=1, keepdims=True)
    keep = jnp.logical_not(supp)
    kf = keep.astype(jnp.float32)
    sel = jnp.where(keep, ii, -1).astype(jnp.float32)
    out_ref[...] = jnp.concatenate(
        [r[:, 0:4] * kf, r[:, 4:5] * kf, r[:, 5:6] * kf, kf, sel], axis=1
    )


def kernel(y_pred):
    p = jnp.reshape(y_pred, (N, PRED))
    p = jnp.pad(p, ((0, NPAD - N), (0, 0)))
    rows = pl.pallas_call(
        _decode_kernel,
        grid=(NPAD // TA,),
        in_specs=[pl.BlockSpec((TA, PRED), lambda i: (i, 0))],
        out_specs=pl.BlockSpec((TA, 8), lambda i: (i, 0)),
        out_shape=jax.ShapeDtypeStruct((NPAD, 8), jnp.float32),
    )(p)
    cols = rows.T
    out = pl.pallas_call(
        _nms_kernel,
        grid=(NPAD // TI,),
        in_specs=[
            pl.BlockSpec((TI, 8), lambda i: (i, 0)),
            pl.BlockSpec((8, NPAD), lambda i: (0, 0)),
        ],
        out_specs=pl.BlockSpec((TI, 8), lambda i: (i, 0)),
        out_shape=jax.ShapeDtypeStruct((NPAD, 8), jnp.float32),
    )(rows, cols)
    boxes = out[:N, 0:4]
    box_scores = out[:N, 4]
    box_classes = out[:N, 5].astype(jnp.int32)
    selected = out[:N, 7].astype(jnp.int32)
    return boxes, box_scores, box_classes, selected


# speedup vs baseline: 1.0281x; 1.0281x over previous
import jax
import jax.numpy as jnp
from jax.experimental import pallas as pl

N = 5000
PRED = 85
NCLS = 80
NPAD = 5120
TI = 256
TA = 640
IOU_THR = 0.5


def _decode_kernel(p_ref, rows_ref, cols_ref):
    p = p_ref[...]
    cx = p[:, 0:1]
    cy = p[:, 1:2]
    w = p[:, 2:3]
    h = p[:, 3:4]
    conf = p[:, 4:5]
    cls = p[:, 5:PRED]
    m = jnp.max(cls, axis=1, keepdims=True)
    iota = jax.lax.broadcasted_iota(jnp.int32, cls.shape, 1)
    amax = jnp.min(jnp.where(cls == m, iota, NCLS), axis=1, keepdims=True)
    x1 = cx - w * 0.5
    y1 = cy - h * 0.5
    x2 = cx + w * 0.5
    y2 = cy + h * 0.5
    score = conf * m
    area = jnp.maximum(x2 - x1, 0.0) * jnp.maximum(y2 - y1, 0.0)
    zero = jnp.zeros_like(score)
    feats = jnp.concatenate(
        [x1, y1, x2, y2, score, amax.astype(jnp.float32), area, zero], axis=1
    )
    gid = pl.program_id(0) * TA + jax.lax.broadcasted_iota(jnp.int32, (TA, 1), 0)
    feats = jnp.where(gid < N, feats, 0.0)
    rows_ref[...] = feats
    cols_ref[...] = feats.T


def _nms_kernel(rows_ref, cols_ref, boxes_ref, scores_ref, classes_ref, sel_ref):
    r = rows_ref[...]
    c = cols_ref[...]
    xi1 = r[:, 0:1]
    yi1 = r[:, 1:2]
    xi2 = r[:, 2:3]
    yi2 = r[:, 3:4]
    si = r[:, 4:5]
    ai = r[:, 6:7]
    xj1 = c[0:1, :]
    yj1 = c[1:2, :]
    xj2 = c[2:3, :]
    yj2 = c[3:4, :]
    sj = c[4:5, :]
    aj = c[6:7, :]
    ix1 = jnp.maximum(xi1, xj1)
    iy1 = jnp.maximum(yi1, yj1)
    ix2 = jnp.minimum(xi2, xj2)
    iy2 = jnp.minimum(yi2, yj2)
    iw = jnp.maximum(ix2 - ix1, 0.0)
    ih = jnp.maximum(iy2 - iy1, 0.0)
    inter = iw * ih
    union = (ai + aj) - inter
    iou = inter / jnp.maximum(union, 1e-9)
    ii = pl.program_id(0) * TI + jax.lax.broadcasted_iota(jnp.int32, (TI, 1), 0)
    jj = jax.lax.broadcasted_iota(jnp.int32, (1, NPAD), 1)
    higher = (sj > si) | ((sj == si) & (jj < ii))
    supp = jnp.any(higher & (iou > IOU_THR), axis=1, keepdims=True)
    keep = jnp.logical_not(supp)
    kf = keep.astype(jnp.float32)
    boxes_ref[...] = r[:, 0:4] * kf
    scores_ref[...] = r[:, 4:5] * kf
    classes_ref[...] = jnp.where(keep, r[:, 5:6], 0.0).astype(jnp.int32)
    sel_ref[...] = jnp.where(keep, ii, -1)


def kernel(y_pred):
    p = jnp.reshape(y_pred, (N, PRED))
    rows, cols = pl.pallas_call(
        _decode_kernel,
        grid=(NPAD // TA,),
        in_specs=[pl.BlockSpec((TA, PRED), lambda i: (i, 0))],
        out_specs=[
            pl.BlockSpec((TA, 8), lambda i: (i, 0)),
            pl.BlockSpec((8, TA), lambda i: (0, i)),
        ],
        out_shape=[
            jax.ShapeDtypeStruct((NPAD, 8), jnp.float32),
            jax.ShapeDtypeStruct((8, NPAD), jnp.float32),
        ],
    )(p)
    boxes, scores, classes, selected = pl.pallas_call(
        _nms_kernel,
        grid=(NPAD // TI,),
        in_specs=[
            pl.BlockSpec((TI, 8), lambda i: (i, 0)),
            pl.BlockSpec((8, NPAD), lambda i: (0, 0)),
        ],
        out_specs=[
            pl.BlockSpec((TI, 4), lambda i: (i, 0)),
            pl.BlockSpec((TI, 1), lambda i: (i, 0)),
            pl.BlockSpec((TI, 1), lambda i: (i, 0)),
            pl.BlockSpec((TI, 1), lambda i: (i, 0)),
        ],
        out_shape=[
            jax.ShapeDtypeStruct((N, 4), jnp.float32),
            jax.ShapeDtypeStruct((N, 1), jnp.float32),
            jax.ShapeDtypeStruct((N, 1), jnp.int32),
            jax.ShapeDtypeStruct((N, 1), jnp.int32),
        ],
    )(rows, cols)
    return (boxes, scores.reshape(N), classes.reshape(N), selected.reshape(N))
